# baseline (speedup 1.0000x reference)
import functools

import jax
import jax.numpy as jnp
from jax import lax
from jax.experimental import pallas as pl
from jax.experimental.pallas import tpu as pltpu
from jax.experimental.pallas import tpu_sc as plsc

N = 10000
E = 320000
D = 128
P = 4
EPS = 1e-5

NC, NS, L = 2, 16, 16
NW = NC * NS
EPW = E // NW
C = 80
NCHUNK = EPW // C
NPAD = 10240
RPT = NPAD // NS
ZR = 128

_sc_mesh = plsc.VectorSubcoreMesh(
    core_axis_name="c", subcore_axis_name="s", num_cores=NC, num_subcores=NS)


def _prep_body(x_ref, pos_ref, wxi_ref, wxj_ref, wd_ref, b1_ref, a_ref, b_ref):
    x = x_ref[...]
    pd = jnp.dot(pos_ref[...], wd_ref[...], preferred_element_type=jnp.float32)
    a_ref[...] = (jnp.dot(x, wxi_ref[...], preferred_element_type=jnp.float32)
                  + pd + b1_ref[...]).astype(jnp.bfloat16)
    b_ref[...] = (jnp.dot(x, wxj_ref[...], preferred_element_type=jnp.float32)
                  - pd).astype(jnp.bfloat16)


_BN = 1000


def _prep(x, pos, wxi, wxj, wd, b1):
    full = lambda shape: pl.BlockSpec(shape, lambda i: (0,) * len(shape))
    return pl.pallas_call(
        _prep_body,
        grid=(N // _BN,),
        in_specs=[
            pl.BlockSpec((_BN, D), lambda i: (i, 0)),
            pl.BlockSpec((_BN, P), lambda i: (i, 0)),
            full((D, D)), full((D, D)), full((P, D)), full((1, D)),
        ],
        out_specs=[pl.BlockSpec((_BN, D), lambda i: (i, 0))] * 2,
        out_shape=[jax.ShapeDtypeStruct((N, D), jnp.bfloat16)] * 2,
    )(x, pos, wxi, wxj, wd, b1)


@functools.partial(
    pl.kernel,
    out_type=[jax.ShapeDtypeStruct((E, D // 2), jnp.int32),
              jax.ShapeDtypeStruct((E, D // 2), jnp.int32),
              jax.ShapeDtypeStruct((E,), jnp.float32)],
    mesh=_sc_mesh,
    scratch_types=[
        pltpu.VMEM((N * P,), jnp.float32),
        pltpu.VMEM((EPW,), jnp.int32),
        pltpu.VMEM((EPW,), jnp.int32),
        [pltpu.VMEM((C, D // 2), jnp.int32)] * 2,
        [pltpu.VMEM((C, D // 2), jnp.int32)] * 2,
        [pltpu.VMEM((C,), jnp.float32)] * 2,
        [pltpu.SemaphoreType.DMA] * 2,
        [pltpu.SemaphoreType.DMA] * 2,
        [pltpu.SemaphoreType.DMA] * 2,
    ],
    compiler_params=pltpu.CompilerParams(needs_layout_passes=False,
                                         use_tc_tiling_on_sc=False),
)
def _sc_gather(a_hbm, b_hbm, posf_hbm, gi_hbm, gj_hbm, ha_hbm, hb_hbm, n2_hbm,
               posf_v, iv_all, jv_all, ra, rb, n2v, sem_a, sem_b, sem_w):
    wid = lax.axis_index("s") * NC + lax.axis_index("c")
    pltpu.sync_copy(posf_hbm, posf_v)
    pltpu.sync_copy(gi_hbm.at[pl.ds(wid * EPW, EPW)], iv_all)
    pltpu.sync_copy(gj_hbm.at[pl.ds(wid * EPW, EPW)], jv_all)

    def _issue(b, k):
        pltpu.async_copy(a_hbm.at[iv_all.at[pl.ds(k * C, C)]], ra[b], sem_a[b])
        pltpu.async_copy(b_hbm.at[jv_all.at[pl.ds(k * C, C)]], rb[b], sem_b[b])

    def _wait_gather(b):
        pltpu.make_async_copy(a_hbm.at[iv_all.at[pl.ds(0, C)]], ra[b], sem_a[b]).wait()
        pltpu.make_async_copy(b_hbm.at[jv_all.at[pl.ds(0, C)]], rb[b], sem_b[b]).wait()

    def _wait_write(b):
        pltpu.make_async_copy(ra[b], ha_hbm.at[pl.ds(0, C)], sem_w[b]).wait()
        pltpu.make_async_copy(rb[b], hb_hbm.at[pl.ds(0, C)], sem_w[b]).wait()
        pltpu.make_async_copy(n2v[b], n2_hbm.at[pl.ds(0, C)], sem_w[b]).wait()

    def _compute(b, k):
        off = k * C
        for g_ in range(C // L):
            ivv = iv_all[pl.ds(off + g_ * L, L)] * P
            jvv = jv_all[pl.ds(off + g_ * L, L)] * P
            acc = jnp.zeros((L,), jnp.float32)
            for comp in range(P):
                pi = plsc.load_gather(posf_v, [ivv + comp])
                pj = plsc.load_gather(posf_v, [jvv + comp])
                dd = pi - pj
                acc = acc + dd * dd
            n2v[b][pl.ds(g_ * L, L)] = acc

        base = wid * EPW + off
        pltpu.async_copy(ra[b], ha_hbm.at[pl.ds(base, C)], sem_w[b])
        pltpu.async_copy(rb[b], hb_hbm.at[pl.ds(base, C)], sem_w[b])
        pltpu.async_copy(n2v[b], n2_hbm.at[pl.ds(base, C)], sem_w[b])

    for b in range(2):
        _issue(b, b)

    @pl.loop(0, (NCHUNK - 1) // 2)
    def _pair(p):
        for b in range(2):
            k = 2 * p + b
            _wait_gather(b)
            _compute(b, k)
            _wait_write(b)
            if b == 0:
                _issue(0, k + 2)
            else:
                @pl.when(p < (NCHUNK - 1) // 2 - 1)
                def _():
                    _issue(1, k + 2)

    _wait_gather(0)
    _compute(0, NCHUNK - 1)
    _wait_write(0)


def _edge_body(ha_ref, hb_ref, n2_ref, wnrm_ref, we2_ref, b2_ref, g_ref,
               bb_ref, o_ref):
    nrm = jnp.sqrt(n2_ref[...])
    h_pre = (ha_ref[...].astype(jnp.float32) + hb_ref[...].astype(jnp.float32))
    h1 = jnp.maximum(h_pre + nrm * wnrm_ref[...], 0.0)
    h2 = jnp.dot(h1, we2_ref[...], preferred_element_type=jnp.float32) + b2_ref[...]
    mu = jnp.mean(h2, axis=1, keepdims=True)
    var = jnp.mean((h2 - mu) ** 2, axis=1, keepdims=True)
    o_ref[...] = (h2 - mu) / jnp.sqrt(var + EPS) * g_ref[...] + bb_ref[...]


_BE = 2000


def _edge_tail(ha, hb, n2, wnrm, we2, b2, g, bb):
    full = lambda shape: pl.BlockSpec(shape, lambda i: (0,) * len(shape))
    return pl.pallas_call(
        _edge_body,
        grid=(E // _BE,),
        in_specs=[
            pl.BlockSpec((_BE, D), lambda i: (i, 0)),
            pl.BlockSpec((_BE, D), lambda i: (i, 0)),
            pl.BlockSpec((_BE, 1), lambda i: (i, 0)),
            full((1, D)), full((D, D)), full((1, D)), full((1, D)), full((1, D)),
        ],
        out_specs=pl.BlockSpec((_BE, D), lambda i: (i, 0)),
        out_shape=jax.ShapeDtypeStruct((E, D), jnp.float32),
    )(ha, hb, n2, wnrm, we2, b2, g, bb)


@functools.partial(
    pl.kernel,
    out_type=jax.ShapeDtypeStruct((NC, NPAD, D), jnp.float32),
    mesh=_sc_mesh,
    scratch_types=[
        pltpu.VMEM_SHARED((NPAD, D), jnp.float32),
        [pltpu.VMEM((C,), jnp.int32)] * 2,
        [pltpu.VMEM((C, D), jnp.float32)] * 2,
        pltpu.VMEM((ZR, D), jnp.float32),
        [pltpu.SemaphoreType.DMA] * 2,
    ],
    compiler_params=pltpu.CompilerParams(needs_layout_passes=False),
)
def _sc_scatter(e_hbm, gj_hbm, out_hbm, aggr_s, jv, ebuf, zbuf, sem_e):
    cid = lax.axis_index("c")
    sid = lax.axis_index("s")
    wid = sid * NC + cid

    @pl.loop(0, ZR)
    def _z(r):
        for cc in range(D // L):
            zbuf[r, pl.ds(cc * L, L)] = jnp.zeros((L,), jnp.float32)

    for q in range(RPT // ZR):
        pltpu.sync_copy(zbuf, aggr_s.at[pl.ds(sid * RPT + q * ZR, ZR)])
    plsc.subcore_barrier()

    def _issue(b, k):
        base = wid * EPW + k * C
        pltpu.sync_copy(gj_hbm.at[pl.ds(base, C)], jv[b])
        pltpu.async_copy(e_hbm.at[pl.ds(base, C)], ebuf[b], sem_e[b])

    def _consume(b):
        pltpu.make_async_copy(e_hbm.at[pl.ds(0, C)], ebuf[b], sem_e[b]).wait()
        pltpu.sync_copy(ebuf[b], aggr_s.at[jv[b]], add=True)

    for b in range(2):
        _issue(b, b)

    @pl.loop(0, (NCHUNK - 1) // 2)
    def _pair(p):
        for b in range(2):
            k = 2 * p + b
            _consume(b)
            if b == 0:
                _issue(0, k + 2)
            else:
                @pl.when(p < (NCHUNK - 1) // 2 - 1)
                def _():
                    _issue(1, k + 2)

    _consume(0)
    plsc.subcore_barrier()
    for q in range(RPT // ZR):
        off = sid * RPT + q * ZR
        pltpu.sync_copy(aggr_s.at[pl.ds(off, ZR)], zbuf)
        pltpu.sync_copy(zbuf, out_hbm.at[cid, pl.ds(off, ZR)])


def _node_body(x_ref, p0_ref, p1_ref, w1x_ref, w1a_ref, b1_ref, w2_ref,
               b2_ref, g_ref, bb_ref, o_ref):
    x = x_ref[...]
    aggr = p0_ref[...] + p1_ref[...]
    h1 = jnp.maximum(
        jnp.dot(x, w1x_ref[...], preferred_element_type=jnp.float32)
        + jnp.dot(aggr, w1a_ref[...], preferred_element_type=jnp.float32)
        + b1_ref[...], 0.0)
    h2 = jnp.dot(h1, w2_ref[...], preferred_element_type=jnp.float32) + b2_ref[...]
    mu = jnp.mean(h2, axis=1, keepdims=True)
    var = jnp.mean((h2 - mu) ** 2, axis=1, keepdims=True)
    o_ref[...] = (h2 - mu) / jnp.sqrt(var + EPS) * g_ref[...] + bb_ref[...] + x


def _node_mlp(x, p0, p1, w1x, w1a, b1, w2, b2, g, bb):
    full = lambda shape: pl.BlockSpec(shape, lambda i: (0,) * len(shape))
    row = pl.BlockSpec((_BN, D), lambda i: (i, 0))
    return pl.pallas_call(
        _node_body,
        grid=(N // _BN,),
        in_specs=[row, row, row,
                  full((D, D)), full((D, D)), full((1, D)), full((D, D)),
                  full((1, D)), full((1, D)), full((1, D))],
        out_specs=row,
        out_shape=jax.ShapeDtypeStruct((N, D), jnp.float32),
    )(x, p0, p1, w1x, w1a, b1, w2, b2, g, bb)


def kernel(x, g, pos, We1, be1, We2, be2, lne_w, lne_b,
           Wn1, bn1, Wn2, bn2, lnn_w, lnn_b):
    r = lambda v: v.reshape(1, D)
    wd = We1[0:P]
    wnrm = We1[P:P + 1]
    wxi = We1[P + 1:P + 1 + D]
    wxj = We1[P + 1 + D:]
    gi = g[0]
    gj = g[1]

    a_tab, b_tab = _prep(x, pos, wxi, wxj, wd, r(be1))
    pack = lambda t: jax.lax.bitcast_convert_type(
        t.reshape(N, D // 2, 2), jnp.int32)
    unpack = lambda t: jax.lax.bitcast_convert_type(
        t, jnp.bfloat16).reshape(E, D)
    ha32, hb32, n2 = _sc_gather(pack(a_tab), pack(b_tab),
                                pos.reshape(-1), gi, gj)
    e_emb = _edge_tail(unpack(ha32), unpack(hb32), n2.reshape(E, 1), wnrm,
                       We2, r(be2), r(lne_w), r(lne_b))
    parts = _sc_scatter(e_emb, gj)
    return _node_mlp(x, parts[0, :N], parts[1, :N], Wn1[:D], Wn1[D:], r(bn1),
                     Wn2, r(bn2), r(lnn_w), r(lnn_b))


# speedup vs baseline: 2.8306x; 2.8306x over previous
import functools

import jax
import jax.numpy as jnp
from jax import lax
from jax.experimental import pallas as pl
from jax.experimental.pallas import tpu as pltpu
from jax.experimental.pallas import tpu_sc as plsc

N = 10000
E = 320000
D = 128
P = 4
EPS = 1e-5

NC, NS, L = 2, 16, 16
NW = NC * NS
EPW = E // NW
C = 80
NCHUNK = EPW // C
NPAD = 10240
RPT = NPAD // NS
ZR = 128

_sc_mesh = plsc.VectorSubcoreMesh(
    core_axis_name="c", subcore_axis_name="s", num_cores=NC, num_subcores=NS)


def _prep_body(x_ref, pos_ref, wxi_ref, wxj_ref, wd_ref, b1_ref, a_ref, b_ref):
    x = x_ref[...]
    pd = jnp.dot(pos_ref[...], wd_ref[...], preferred_element_type=jnp.float32)
    a_ref[...] = (jnp.dot(x, wxi_ref[...], preferred_element_type=jnp.float32)
                  + pd + b1_ref[...]).astype(jnp.bfloat16)
    b_ref[...] = (jnp.dot(x, wxj_ref[...], preferred_element_type=jnp.float32)
                  - pd).astype(jnp.bfloat16)


_BN = 1000


def _prep(x, pos, wxi, wxj, wd, b1):
    full = lambda shape: pl.BlockSpec(shape, lambda i: (0,) * len(shape))
    return pl.pallas_call(
        _prep_body,
        grid=(N // _BN,),
        in_specs=[
            pl.BlockSpec((_BN, D), lambda i: (i, 0)),
            pl.BlockSpec((_BN, P), lambda i: (i, 0)),
            full((D, D)), full((D, D)), full((P, D)), full((1, D)),
        ],
        out_specs=[pl.BlockSpec((_BN, D), lambda i: (i, 0))] * 2,
        out_shape=[jax.ShapeDtypeStruct((N, D), jnp.bfloat16)] * 2,
    )(x, pos, wxi, wxj, wd, b1)


@functools.partial(
    pl.kernel,
    out_type=[jax.ShapeDtypeStruct((E, D), jnp.int32),
              jax.ShapeDtypeStruct((E,), jnp.float32)],
    mesh=_sc_mesh,
    scratch_types=[
        pltpu.VMEM((N * P,), jnp.float32),
        pltpu.VMEM((EPW,), jnp.int32),
        pltpu.VMEM((EPW,), jnp.int32),
        [pltpu.VMEM((C, D // 2), jnp.int32)] * 2,
        [pltpu.VMEM((C, D // 2), jnp.int32)] * 2,
        [pltpu.VMEM((C,), jnp.float32)] * 2,
        [pltpu.SemaphoreType.DMA] * 2,
        [pltpu.SemaphoreType.DMA] * 2,
        [pltpu.SemaphoreType.DMA] * 2,
    ],
    compiler_params=pltpu.CompilerParams(needs_layout_passes=False,
                                         use_tc_tiling_on_sc=False),
)
def _sc_gather(a_hbm, b_hbm, posf_hbm, gi_hbm, gj_hbm, hab_hbm, n2_hbm,
               posf_v, iv_all, jv_all, ra, rb, n2v, sem_a, sem_b, sem_w):
    wid = lax.axis_index("s") * NC + lax.axis_index("c")
    pltpu.sync_copy(posf_hbm, posf_v)
    pltpu.sync_copy(gi_hbm.at[pl.ds(wid * EPW, EPW)], iv_all)
    pltpu.sync_copy(gj_hbm.at[pl.ds(wid * EPW, EPW)], jv_all)

    def _issue(b, k):
        pltpu.async_copy(a_hbm.at[iv_all.at[pl.ds(k * C, C)]], ra[b], sem_a[b])
        pltpu.async_copy(b_hbm.at[jv_all.at[pl.ds(k * C, C)]], rb[b], sem_b[b])

    def _wait_gather(b):
        pltpu.make_async_copy(a_hbm.at[iv_all.at[pl.ds(0, C)]], ra[b], sem_a[b]).wait()
        pltpu.make_async_copy(b_hbm.at[jv_all.at[pl.ds(0, C)]], rb[b], sem_b[b]).wait()

    def _wait_write(b):
        pltpu.make_async_copy(
            ra[b], hab_hbm.at[pl.ds(0, C), pl.ds(0, D // 2)], sem_w[b]).wait()
        pltpu.make_async_copy(
            rb[b], hab_hbm.at[pl.ds(0, C), pl.ds(D // 2, D // 2)], sem_w[b]).wait()
        pltpu.make_async_copy(n2v[b], n2_hbm.at[pl.ds(0, C)], sem_w[b]).wait()

    def _compute(b, k):
        off = k * C
        for g_ in range(C // L):
            ivv = iv_all[pl.ds(off + g_ * L, L)] * P
            jvv = jv_all[pl.ds(off + g_ * L, L)] * P
            acc = jnp.zeros((L,), jnp.float32)
            for comp in range(P):
                pi = plsc.load_gather(posf_v, [ivv + comp])
                pj = plsc.load_gather(posf_v, [jvv + comp])
                dd = pi - pj
                acc = acc + dd * dd
            n2v[b][pl.ds(g_ * L, L)] = acc

        base = wid * EPW + off
        pltpu.async_copy(
            ra[b], hab_hbm.at[pl.ds(base, C), pl.ds(0, D // 2)], sem_w[b])
        pltpu.async_copy(
            rb[b], hab_hbm.at[pl.ds(base, C), pl.ds(D // 2, D // 2)], sem_w[b])
        pltpu.async_copy(n2v[b], n2_hbm.at[pl.ds(base, C)], sem_w[b])

    for b in range(2):
        _issue(b, b)

    @pl.loop(0, (NCHUNK - 1) // 2)
    def _pair(p):
        for b in range(2):
            k = 2 * p + b
            _wait_gather(b)
            _compute(b, k)
            _wait_write(b)
            if b == 0:
                _issue(0, k + 2)
            else:
                @pl.when(p < (NCHUNK - 1) // 2 - 1)
                def _():
                    _issue(1, k + 2)

    _wait_gather(0)
    _compute(0, NCHUNK - 1)
    _wait_write(0)


def _edge_body(hab_ref, n2_ref, wnrm_ref, we2_ref, b2_ref, g_ref,
               bb_ref, o_ref):
    w = hab_ref[...]
    a32 = w[:, :D // 2]
    b32 = w[:, D // 2:]
    asf = lambda v: jax.lax.bitcast_convert_type(v, jnp.float32)
    lo = lambda v: asf(jax.lax.shift_left(v, 16))
    hi = lambda v: asf(jax.lax.bitwise_and(v, jnp.int32(-65536)))
    h_pre = jnp.concatenate(
        [lo(a32) + lo(b32), hi(a32) + hi(b32)], axis=1)
    nrm = jnp.sqrt(n2_ref[...])
    h1 = jnp.maximum(h_pre + nrm * wnrm_ref[...], 0.0)
    h2 = jnp.dot(h1, we2_ref[...], preferred_element_type=jnp.float32) + b2_ref[...]
    mu = jnp.mean(h2, axis=1, keepdims=True)
    var = jnp.mean((h2 - mu) ** 2, axis=1, keepdims=True)
    o_ref[...] = (h2 - mu) / jnp.sqrt(var + EPS) * g_ref[...] + bb_ref[...]


_BE = 2000


def _edge_tail(hab, n2, wnrm, we2, b2, g, bb):
    full = lambda shape: pl.BlockSpec(shape, lambda i: (0,) * len(shape))
    return pl.pallas_call(
        _edge_body,
        grid=(E // _BE,),
        in_specs=[
            pl.BlockSpec((_BE, D), lambda i: (i, 0)),
            pl.BlockSpec((_BE, 1), lambda i: (i, 0)),
            full((1, D)), full((D, D)), full((1, D)), full((1, D)), full((1, D)),
        ],
        out_specs=pl.BlockSpec((_BE, D), lambda i: (i, 0)),
        out_shape=jax.ShapeDtypeStruct((E, D), jnp.float32),
    )(hab, n2, wnrm, we2, b2, g, bb)


@functools.partial(
    pl.kernel,
    out_type=jax.ShapeDtypeStruct((NC, NPAD, D), jnp.float32),
    mesh=_sc_mesh,
    scratch_types=[
        pltpu.VMEM_SHARED((NPAD, D), jnp.float32),
        [pltpu.VMEM((C,), jnp.int32)] * 2,
        [pltpu.VMEM((C, D), jnp.float32)] * 2,
        pltpu.VMEM((ZR, D), jnp.float32),
        [pltpu.SemaphoreType.DMA] * 2,
    ],
    compiler_params=pltpu.CompilerParams(needs_layout_passes=False),
)
def _sc_scatter(e_hbm, gj_hbm, out_hbm, aggr_s, jv, ebuf, zbuf, sem_e):
    cid = lax.axis_index("c")
    sid = lax.axis_index("s")
    wid = sid * NC + cid

    @pl.loop(0, ZR)
    def _z(r):
        for cc in range(D // L):
            zbuf[r, pl.ds(cc * L, L)] = jnp.zeros((L,), jnp.float32)

    for q in range(RPT // ZR):
        pltpu.sync_copy(zbuf, aggr_s.at[pl.ds(sid * RPT + q * ZR, ZR)])
    plsc.subcore_barrier()

    def _issue(b, k):
        base = wid * EPW + k * C
        pltpu.sync_copy(gj_hbm.at[pl.ds(base, C)], jv[b])
        pltpu.async_copy(e_hbm.at[pl.ds(base, C)], ebuf[b], sem_e[b])

    def _consume(b):
        pltpu.make_async_copy(e_hbm.at[pl.ds(0, C)], ebuf[b], sem_e[b]).wait()
        pltpu.sync_copy(ebuf[b], aggr_s.at[jv[b]], add=True)

    for b in range(2):
        _issue(b, b)

    @pl.loop(0, (NCHUNK - 1) // 2)
    def _pair(p):
        for b in range(2):
            k = 2 * p + b
            _consume(b)
            if b == 0:
                _issue(0, k + 2)
            else:
                @pl.when(p < (NCHUNK - 1) // 2 - 1)
                def _():
                    _issue(1, k + 2)

    _consume(0)
    plsc.subcore_barrier()
    for q in range(RPT // ZR):
        off = sid * RPT + q * ZR
        pltpu.sync_copy(aggr_s.at[pl.ds(off, ZR)], zbuf)
        pltpu.sync_copy(zbuf, out_hbm.at[cid, pl.ds(off, ZR)])


def _node_body(x_ref, p0_ref, p1_ref, w1x_ref, w1a_ref, b1_ref, w2_ref,
               b2_ref, g_ref, bb_ref, o_ref):
    x = x_ref[...]
    aggr = p0_ref[...] + p1_ref[...]
    h1 = jnp.maximum(
        jnp.dot(x, w1x_ref[...], preferred_element_type=jnp.float32)
        + jnp.dot(aggr, w1a_ref[...], preferred_element_type=jnp.float32)
        + b1_ref[...], 0.0)
    h2 = jnp.dot(h1, w2_ref[...], preferred_element_type=jnp.float32) + b2_ref[...]
    mu = jnp.mean(h2, axis=1, keepdims=True)
    var = jnp.mean((h2 - mu) ** 2, axis=1, keepdims=True)
    o_ref[...] = (h2 - mu) / jnp.sqrt(var + EPS) * g_ref[...] + bb_ref[...] + x


def _node_mlp(x, p0, p1, w1x, w1a, b1, w2, b2, g, bb):
    full = lambda shape: pl.BlockSpec(shape, lambda i: (0,) * len(shape))
    row = pl.BlockSpec((_BN, D), lambda i: (i, 0))
    return pl.pallas_call(
        _node_body,
        grid=(N // _BN,),
        in_specs=[row, row, row,
                  full((D, D)), full((D, D)), full((1, D)), full((D, D)),
                  full((1, D)), full((1, D)), full((1, D))],
        out_specs=row,
        out_shape=jax.ShapeDtypeStruct((N, D), jnp.float32),
    )(x, p0, p1, w1x, w1a, b1, w2, b2, g, bb)


def kernel(x, g, pos, We1, be1, We2, be2, lne_w, lne_b,
           Wn1, bn1, Wn2, bn2, lnn_w, lnn_b):
    r = lambda v: v.reshape(1, D)
    wd = We1[0:P]
    wnrm = We1[P:P + 1]
    wxi = We1[P + 1:P + 1 + D]
    wxj = We1[P + 1 + D:]
    gi = g[0]
    gj = g[1]

    a_tab, b_tab = _prep(x, pos, wxi, wxj, wd, r(be1))
    pack = lambda t: jax.lax.bitcast_convert_type(
        t.reshape(N, D // 2, 2), jnp.int32)
    hab, n2 = _sc_gather(pack(a_tab), pack(b_tab), pos.reshape(-1), gi, gj)
    perm = jnp.arange(D).reshape(D // 2, 2).T.reshape(D)
    e_emb = _edge_tail(hab, n2.reshape(E, 1), wnrm[:, perm], We2[perm],
                       r(be2), r(lne_w), r(lne_b))
    parts = _sc_scatter(e_emb, gj)
    return _node_mlp(x, parts[0, :N], parts[1, :N], Wn1[:D], Wn1[D:], r(bn1),
                     Wn2, r(bn2), r(lnn_w), r(lnn_b))


# speedup vs baseline: 2.8345x; 1.0014x over previous
import functools

import jax
import jax.numpy as jnp
from jax import lax
from jax.experimental import pallas as pl
from jax.experimental.pallas import tpu as pltpu
from jax.experimental.pallas import tpu_sc as plsc

N = 10000
E = 320000
D = 128
P = 4
EPS = 1e-5

NC, NS, L = 2, 16, 16
NW = NC * NS
EPW = E // NW
C = 80
NCHUNK = EPW // C
NPAD = 10240
RPT = NPAD // NS
ZR = 128

_sc_mesh = plsc.VectorSubcoreMesh(
    core_axis_name="c", subcore_axis_name="s", num_cores=NC, num_subcores=NS)


def _prep_body(x_ref, pos_ref, wxi_ref, wxj_ref, wd_ref, b1_ref, a_ref, b_ref):
    x = x_ref[...]
    pd = jnp.dot(pos_ref[...], wd_ref[...], preferred_element_type=jnp.float32)
    a_ref[...] = (jnp.dot(x, wxi_ref[...], preferred_element_type=jnp.float32)
                  + pd + b1_ref[...]).astype(jnp.bfloat16)
    b_ref[...] = (jnp.dot(x, wxj_ref[...], preferred_element_type=jnp.float32)
                  - pd).astype(jnp.bfloat16)


_BN = 1000


def _prep(x, pos, wxi, wxj, wd, b1):
    full = lambda shape: pl.BlockSpec(shape, lambda i: (0,) * len(shape))
    return pl.pallas_call(
        _prep_body,
        grid=(N // _BN,),
        in_specs=[
            pl.BlockSpec((_BN, D), lambda i: (i, 0)),
            pl.BlockSpec((_BN, P), lambda i: (i, 0)),
            full((D, D)), full((D, D)), full((P, D)), full((1, D)),
        ],
        out_specs=[pl.BlockSpec((_BN, D), lambda i: (i, 0))] * 2,
        out_shape=[jax.ShapeDtypeStruct((N, D), jnp.bfloat16)] * 2,
    )(x, pos, wxi, wxj, wd, b1)


@functools.partial(
    pl.kernel,
    out_type=[jax.ShapeDtypeStruct((E, D), jnp.int32),
              jax.ShapeDtypeStruct((E,), jnp.float32)],
    mesh=_sc_mesh,
    scratch_types=[
        pltpu.VMEM((N * P,), jnp.float32),
        pltpu.VMEM((EPW,), jnp.int32),
        pltpu.VMEM((EPW,), jnp.int32),
        [pltpu.VMEM((C, D // 2), jnp.int32)] * 2,
        [pltpu.VMEM((C, D // 2), jnp.int32)] * 2,
        [pltpu.VMEM((C,), jnp.float32)] * 2,
        [pltpu.SemaphoreType.DMA] * 2,
        [pltpu.SemaphoreType.DMA] * 2,
        [pltpu.SemaphoreType.DMA] * 2,
    ],
    compiler_params=pltpu.CompilerParams(needs_layout_passes=False,
                                         use_tc_tiling_on_sc=False),
)
def _sc_gather(a_hbm, b_hbm, posf_hbm, gi_hbm, gj_hbm, hab_hbm, n2_hbm,
               posf_v, iv_all, jv_all, ra, rb, n2v, sem_a, sem_b, sem_w):
    wid = lax.axis_index("s") * NC + lax.axis_index("c")
    pltpu.sync_copy(posf_hbm, posf_v)
    pltpu.sync_copy(gi_hbm.at[pl.ds(wid * EPW, EPW)], iv_all)
    pltpu.sync_copy(gj_hbm.at[pl.ds(wid * EPW, EPW)], jv_all)

    def _issue(b, k):
        pltpu.async_copy(a_hbm.at[iv_all.at[pl.ds(k * C, C)]], ra[b], sem_a[b])
        pltpu.async_copy(b_hbm.at[jv_all.at[pl.ds(k * C, C)]], rb[b], sem_b[b])

    def _wait_gather(b):
        pltpu.make_async_copy(a_hbm.at[iv_all.at[pl.ds(0, C)]], ra[b], sem_a[b]).wait()
        pltpu.make_async_copy(b_hbm.at[jv_all.at[pl.ds(0, C)]], rb[b], sem_b[b]).wait()

    def _wait_write(b):
        pltpu.make_async_copy(
            ra[b], hab_hbm.at[pl.ds(0, C), pl.ds(0, D // 2)], sem_w[b]).wait()
        pltpu.make_async_copy(
            rb[b], hab_hbm.at[pl.ds(0, C), pl.ds(D // 2, D // 2)], sem_w[b]).wait()
        pltpu.make_async_copy(n2v[b], n2_hbm.at[pl.ds(0, C)], sem_w[b]).wait()

    def _compute(b, k):
        off = k * C
        for g_ in range(C // L):
            ivv = iv_all[pl.ds(off + g_ * L, L)] * P
            jvv = jv_all[pl.ds(off + g_ * L, L)] * P
            acc = jnp.zeros((L,), jnp.float32)
            for comp in range(P):
                pi = plsc.load_gather(posf_v, [ivv + comp])
                pj = plsc.load_gather(posf_v, [jvv + comp])
                dd = pi - pj
                acc = acc + dd * dd
            n2v[b][pl.ds(g_ * L, L)] = acc

        base = wid * EPW + off
        pltpu.async_copy(
            ra[b], hab_hbm.at[pl.ds(base, C), pl.ds(0, D // 2)], sem_w[b])
        pltpu.async_copy(
            rb[b], hab_hbm.at[pl.ds(base, C), pl.ds(D // 2, D // 2)], sem_w[b])
        pltpu.async_copy(n2v[b], n2_hbm.at[pl.ds(base, C)], sem_w[b])

    for b in range(2):
        _issue(b, b)

    @pl.loop(0, (NCHUNK - 1) // 2)
    def _pair(p):
        for b in range(2):
            k = 2 * p + b
            _wait_gather(b)
            _compute(b, k)
            _wait_write(b)
            if b == 0:
                _issue(0, k + 2)
            else:
                @pl.when(p < (NCHUNK - 1) // 2 - 1)
                def _():
                    _issue(1, k + 2)

    _wait_gather(0)
    _compute(0, NCHUNK - 1)
    _wait_write(0)


def _edge_body(hab_ref, n2_ref, wnrm_ref, we2_ref, b2_ref, g_ref,
               bb_ref, o_ref):
    w = hab_ref[...]
    a32 = w[:, :D // 2]
    b32 = w[:, D // 2:]
    asf = lambda v: jax.lax.bitcast_convert_type(v, jnp.float32)
    lo = lambda v: asf(jax.lax.shift_left(v, 16))
    hi = lambda v: asf(jax.lax.bitwise_and(v, jnp.int32(-65536)))
    h_pre = jnp.concatenate(
        [lo(a32) + lo(b32), hi(a32) + hi(b32)], axis=1)
    nrm = jnp.sqrt(n2_ref[...])
    h1 = jnp.maximum(h_pre + nrm * wnrm_ref[...], 0.0).astype(jnp.bfloat16)
    h2 = jnp.dot(h1, we2_ref[...], preferred_element_type=jnp.float32) + b2_ref[...]
    mu = jnp.mean(h2, axis=1, keepdims=True)
    var = jnp.mean((h2 - mu) ** 2, axis=1, keepdims=True)
    o_ref[...] = (h2 - mu) / jnp.sqrt(var + EPS) * g_ref[...] + bb_ref[...]


_BE = 2000


def _edge_tail(hab, n2, wnrm, we2, b2, g, bb):
    full = lambda shape: pl.BlockSpec(shape, lambda i: (0,) * len(shape))
    return pl.pallas_call(
        _edge_body,
        grid=(E // _BE,),
        in_specs=[
            pl.BlockSpec((_BE, D), lambda i: (i, 0)),
            pl.BlockSpec((_BE, 1), lambda i: (i, 0)),
            full((1, D)), full((D, D)), full((1, D)), full((1, D)), full((1, D)),
        ],
        out_specs=pl.BlockSpec((_BE, D), lambda i: (i, 0)),
        out_shape=jax.ShapeDtypeStruct((E, D), jnp.float32),
    )(hab, n2, wnrm, we2, b2, g, bb)


@functools.partial(
    pl.kernel,
    out_type=jax.ShapeDtypeStruct((NC, NPAD, D), jnp.float32),
    mesh=_sc_mesh,
    scratch_types=[
        pltpu.VMEM_SHARED((NPAD, D), jnp.float32),
        [pltpu.VMEM((C,), jnp.int32)] * 2,
        [pltpu.VMEM((C, D), jnp.float32)] * 2,
        pltpu.VMEM((ZR, D), jnp.float32),
        [pltpu.SemaphoreType.DMA] * 2,
    ],
    compiler_params=pltpu.CompilerParams(needs_layout_passes=False),
)
def _sc_scatter(e_hbm, gj_hbm, out_hbm, aggr_s, jv, ebuf, zbuf, sem_e):
    cid = lax.axis_index("c")
    sid = lax.axis_index("s")
    wid = sid * NC + cid

    @pl.loop(0, ZR)
    def _z(r):
        for cc in range(D // L):
            zbuf[r, pl.ds(cc * L, L)] = jnp.zeros((L,), jnp.float32)

    for q in range(RPT // ZR):
        pltpu.sync_copy(zbuf, aggr_s.at[pl.ds(sid * RPT + q * ZR, ZR)])
    plsc.subcore_barrier()

    def _issue(b, k):
        base = wid * EPW + k * C
        pltpu.sync_copy(gj_hbm.at[pl.ds(base, C)], jv[b])
        pltpu.async_copy(e_hbm.at[pl.ds(base, C)], ebuf[b], sem_e[b])

    def _consume(b):
        pltpu.make_async_copy(e_hbm.at[pl.ds(0, C)], ebuf[b], sem_e[b]).wait()
        pltpu.sync_copy(ebuf[b], aggr_s.at[jv[b]], add=True)

    for b in range(2):
        _issue(b, b)

    @pl.loop(0, (NCHUNK - 1) // 2)
    def _pair(p):
        for b in range(2):
            k = 2 * p + b
            _consume(b)
            if b == 0:
                _issue(0, k + 2)
            else:
                @pl.when(p < (NCHUNK - 1) // 2 - 1)
                def _():
                    _issue(1, k + 2)

    _consume(0)
    plsc.subcore_barrier()
    for q in range(RPT // ZR):
        off = sid * RPT + q * ZR
        pltpu.sync_copy(aggr_s.at[pl.ds(off, ZR)], zbuf)
        pltpu.sync_copy(zbuf, out_hbm.at[cid, pl.ds(off, ZR)])


def _node_body(x_ref, p0_ref, p1_ref, w1x_ref, w1a_ref, b1_ref, w2_ref,
               b2_ref, g_ref, bb_ref, o_ref):
    x = x_ref[...]
    aggr = p0_ref[...] + p1_ref[...]
    h1 = jnp.maximum(
        jnp.dot(x, w1x_ref[...], preferred_element_type=jnp.float32)
        + jnp.dot(aggr, w1a_ref[...], preferred_element_type=jnp.float32)
        + b1_ref[...], 0.0)
    h2 = jnp.dot(h1, w2_ref[...], preferred_element_type=jnp.float32) + b2_ref[...]
    mu = jnp.mean(h2, axis=1, keepdims=True)
    var = jnp.mean((h2 - mu) ** 2, axis=1, keepdims=True)
    o_ref[...] = (h2 - mu) / jnp.sqrt(var + EPS) * g_ref[...] + bb_ref[...] + x


def _node_mlp(x, p0, p1, w1x, w1a, b1, w2, b2, g, bb):
    full = lambda shape: pl.BlockSpec(shape, lambda i: (0,) * len(shape))
    row = pl.BlockSpec((_BN, D), lambda i: (i, 0))
    return pl.pallas_call(
        _node_body,
        grid=(N // _BN,),
        in_specs=[row, row, row,
                  full((D, D)), full((D, D)), full((1, D)), full((D, D)),
                  full((1, D)), full((1, D)), full((1, D))],
        out_specs=row,
        out_shape=jax.ShapeDtypeStruct((N, D), jnp.float32),
    )(x, p0, p1, w1x, w1a, b1, w2, b2, g, bb)


def kernel(x, g, pos, We1, be1, We2, be2, lne_w, lne_b,
           Wn1, bn1, Wn2, bn2, lnn_w, lnn_b):
    r = lambda v: v.reshape(1, D)
    wd = We1[0:P]
    wnrm = We1[P:P + 1]
    wxi = We1[P + 1:P + 1 + D]
    wxj = We1[P + 1 + D:]
    gi = g[0]
    gj = g[1]

    a_tab, b_tab = _prep(x, pos, wxi, wxj, wd, r(be1))
    pack = lambda t: jax.lax.bitcast_convert_type(
        t.reshape(N, D // 2, 2), jnp.int32)
    hab, n2 = _sc_gather(pack(a_tab), pack(b_tab), pos.reshape(-1), gi, gj)
    perm = jnp.arange(D).reshape(D // 2, 2).T.reshape(D)
    e_emb = _edge_tail(hab, n2.reshape(E, 1), wnrm[:, perm],
                       We2[perm].astype(jnp.bfloat16),
                       r(be2), r(lne_w), r(lne_b))
    parts = _sc_scatter(e_emb, gj)
    return _node_mlp(x, parts[0, :N], parts[1, :N], Wn1[:D], Wn1[D:], r(bn1),
                     Wn2, r(bn2), r(lnn_w), r(lnn_b))


# speedup vs baseline: 2.9135x; 1.0279x over previous
import functools

import jax
import jax.numpy as jnp
from jax import lax
from jax.experimental import pallas as pl
from jax.experimental.pallas import tpu as pltpu
from jax.experimental.pallas import tpu_sc as plsc

N = 10000
E = 320000
D = 128
P = 4
EPS = 1e-5

NC, NS, L = 2, 16, 16
NW = NC * NS
EPW = E // NW
C = 80
NCHUNK = EPW // C
NPAD = 10240
RPT = NPAD // NS
ZR = 128

_sc_mesh = plsc.VectorSubcoreMesh(
    core_axis_name="c", subcore_axis_name="s", num_cores=NC, num_subcores=NS)


def _prep_body(x_ref, pos_ref, wxi_ref, wxj_ref, wd_ref, b1_ref, a_ref, b_ref):
    x = x_ref[...]
    pd = jnp.dot(pos_ref[...], wd_ref[...], preferred_element_type=jnp.float32)
    a_ref[...] = (jnp.dot(x, wxi_ref[...], preferred_element_type=jnp.float32)
                  + pd + b1_ref[...]).astype(jnp.bfloat16)
    b_ref[...] = (jnp.dot(x, wxj_ref[...], preferred_element_type=jnp.float32)
                  - pd).astype(jnp.bfloat16)


_BN = 1000


def _prep(x, pos, wxi, wxj, wd, b1):
    full = lambda shape: pl.BlockSpec(shape, lambda i: (0,) * len(shape))
    return pl.pallas_call(
        _prep_body,
        grid=(N // _BN,),
        in_specs=[
            pl.BlockSpec((_BN, D), lambda i: (i, 0)),
            pl.BlockSpec((_BN, P), lambda i: (i, 0)),
            full((D, D)), full((D, D)), full((P, D)), full((1, D)),
        ],
        out_specs=[pl.BlockSpec((_BN, D), lambda i: (i, 0))] * 2,
        out_shape=[jax.ShapeDtypeStruct((N, D), jnp.bfloat16)] * 2,
    )(x, pos, wxi, wxj, wd, b1)


@functools.partial(
    pl.kernel,
    out_type=[jax.ShapeDtypeStruct((E, D), jnp.int32),
              jax.ShapeDtypeStruct((E,), jnp.float32)],
    mesh=_sc_mesh,
    scratch_types=[
        pltpu.VMEM((N * P,), jnp.float32),
        pltpu.VMEM((EPW,), jnp.int32),
        pltpu.VMEM((EPW,), jnp.int32),
        [pltpu.VMEM((C, D // 2), jnp.int32)] * 2,
        [pltpu.VMEM((C, D // 2), jnp.int32)] * 2,
        [pltpu.VMEM((C,), jnp.float32)] * 2,
        [pltpu.SemaphoreType.DMA] * 2,
        [pltpu.SemaphoreType.DMA] * 2,
        [pltpu.SemaphoreType.DMA] * 2,
    ],
    compiler_params=pltpu.CompilerParams(needs_layout_passes=False,
                                         use_tc_tiling_on_sc=False),
)
def _sc_gather(a_hbm, b_hbm, posf_hbm, gi_hbm, gj_hbm, hab_hbm, n2_hbm,
               posf_v, iv_all, jv_all, ra, rb, n2v, sem_a, sem_b, sem_w):
    wid = lax.axis_index("s") * NC + lax.axis_index("c")
    pltpu.sync_copy(posf_hbm, posf_v)
    pltpu.sync_copy(gi_hbm.at[pl.ds(wid * EPW, EPW)], iv_all)
    pltpu.sync_copy(gj_hbm.at[pl.ds(wid * EPW, EPW)], jv_all)

    def _issue(b, k):
        pltpu.async_copy(a_hbm.at[iv_all.at[pl.ds(k * C, C)]], ra[b], sem_a[b])
        pltpu.async_copy(b_hbm.at[jv_all.at[pl.ds(k * C, C)]], rb[b], sem_b[b])

    def _wait_gather(b):
        pltpu.make_async_copy(a_hbm.at[iv_all.at[pl.ds(0, C)]], ra[b], sem_a[b]).wait()
        pltpu.make_async_copy(b_hbm.at[jv_all.at[pl.ds(0, C)]], rb[b], sem_b[b]).wait()

    def _wait_write(b):
        pltpu.make_async_copy(
            ra[b], hab_hbm.at[pl.ds(0, C), pl.ds(0, D // 2)], sem_w[b]).wait()
        pltpu.make_async_copy(
            rb[b], hab_hbm.at[pl.ds(0, C), pl.ds(D // 2, D // 2)], sem_w[b]).wait()
        pltpu.make_async_copy(n2v[b], n2_hbm.at[pl.ds(0, C)], sem_w[b]).wait()

    def _compute(b, k):
        off = k * C
        for g_ in range(C // L):
            ivv = iv_all[pl.ds(off + g_ * L, L)] * P
            jvv = jv_all[pl.ds(off + g_ * L, L)] * P
            acc = jnp.zeros((L,), jnp.float32)
            for comp in range(P):
                pi = plsc.load_gather(posf_v, [ivv + comp])
                pj = plsc.load_gather(posf_v, [jvv + comp])
                dd = pi - pj
                acc = acc + dd * dd
            n2v[b][pl.ds(g_ * L, L)] = acc

        base = wid * EPW + off
        pltpu.async_copy(
            ra[b], hab_hbm.at[pl.ds(base, C), pl.ds(0, D // 2)], sem_w[b])
        pltpu.async_copy(
            rb[b], hab_hbm.at[pl.ds(base, C), pl.ds(D // 2, D // 2)], sem_w[b])
        pltpu.async_copy(n2v[b], n2_hbm.at[pl.ds(base, C)], sem_w[b])

    for b in range(2):
        _issue(b, b)

    @pl.loop(0, (NCHUNK - 1) // 2)
    def _pair(p):
        for b in range(2):
            k = 2 * p + b
            _wait_gather(b)
            _compute(b, k)
            _wait_write(b)
            if b == 0:
                _issue(0, k + 2)
            else:
                @pl.when(p < (NCHUNK - 1) // 2 - 1)
                def _():
                    _issue(1, k + 2)

    _wait_gather(0)
    _compute(0, NCHUNK - 1)
    _wait_write(0)


def _edge_body(hab_ref, n2_ref, wnrm_ref, we2_ref, b2_ref, g_ref,
               bb_ref, o_ref):
    w = hab_ref[...]
    a32 = w[:, :D // 2]
    b32 = w[:, D // 2:]
    asf = lambda v: jax.lax.bitcast_convert_type(v, jnp.float32)
    lo = lambda v: asf(jax.lax.shift_left(v, 16))
    hi = lambda v: asf(jax.lax.bitwise_and(v, jnp.int32(-65536)))
    h_pre = jnp.concatenate(
        [lo(a32) + lo(b32), hi(a32) + hi(b32)], axis=1)
    nrm = jnp.sqrt(n2_ref[...])
    h1 = jnp.maximum(h_pre + nrm * wnrm_ref[...], 0.0)
    h2 = jnp.dot(h1, we2_ref[...], preferred_element_type=jnp.float32) + b2_ref[...]
    mu = jnp.mean(h2, axis=1, keepdims=True)
    var = jnp.mean((h2 - mu) ** 2, axis=1, keepdims=True)
    o_ref[...] = (h2 - mu) / jnp.sqrt(var + EPS) * g_ref[...] + bb_ref[...]


_BE = 4000


def _edge_tail(hab, n2, wnrm, we2, b2, g, bb):
    full = lambda shape: pl.BlockSpec(shape, lambda i: (0,) * len(shape))
    return pl.pallas_call(
        _edge_body,
        grid=(E // _BE,),
        in_specs=[
            pl.BlockSpec((_BE, D), lambda i: (i, 0)),
            pl.BlockSpec((_BE, 1), lambda i: (i, 0)),
            full((1, D)), full((D, D)), full((1, D)), full((1, D)), full((1, D)),
        ],
        out_specs=pl.BlockSpec((_BE, D), lambda i: (i, 0)),
        out_shape=jax.ShapeDtypeStruct((E, D), jnp.float32),
    )(hab, n2, wnrm, we2, b2, g, bb)


@functools.partial(
    pl.kernel,
    out_type=jax.ShapeDtypeStruct((NC, NPAD, D), jnp.float32),
    mesh=_sc_mesh,
    scratch_types=[
        pltpu.VMEM_SHARED((NPAD, D), jnp.float32),
        [pltpu.VMEM((C,), jnp.int32)] * 2,
        [pltpu.VMEM((C, D), jnp.float32)] * 2,
        pltpu.VMEM((ZR, D), jnp.float32),
        [pltpu.SemaphoreType.DMA] * 2,
    ],
    compiler_params=pltpu.CompilerParams(needs_layout_passes=False),
)
def _sc_scatter(e_hbm, gj_hbm, out_hbm, aggr_s, jv, ebuf, zbuf, sem_e):
    cid = lax.axis_index("c")
    sid = lax.axis_index("s")
    wid = sid * NC + cid

    @pl.loop(0, ZR)
    def _z(r):
        for cc in range(D // L):
            zbuf[r, pl.ds(cc * L, L)] = jnp.zeros((L,), jnp.float32)

    for q in range(RPT // ZR):
        pltpu.sync_copy(zbuf, aggr_s.at[pl.ds(sid * RPT + q * ZR, ZR)])
    plsc.subcore_barrier()

    def _issue(b, k):
        base = wid * EPW + k * C
        pltpu.sync_copy(gj_hbm.at[pl.ds(base, C)], jv[b])
        pltpu.async_copy(e_hbm.at[pl.ds(base, C)], ebuf[b], sem_e[b])

    def _consume(b):
        pltpu.make_async_copy(e_hbm.at[pl.ds(0, C)], ebuf[b], sem_e[b]).wait()
        pltpu.sync_copy(ebuf[b], aggr_s.at[jv[b]], add=True)

    for b in range(2):
        _issue(b, b)

    @pl.loop(0, (NCHUNK - 1) // 2)
    def _pair(p):
        for b in range(2):
            k = 2 * p + b
            _consume(b)
            if b == 0:
                _issue(0, k + 2)
            else:
                @pl.when(p < (NCHUNK - 1) // 2 - 1)
                def _():
                    _issue(1, k + 2)

    _consume(0)
    plsc.subcore_barrier()
    for q in range(RPT // ZR):
        off = sid * RPT + q * ZR
        pltpu.sync_copy(aggr_s.at[pl.ds(off, ZR)], zbuf)
        pltpu.sync_copy(zbuf, out_hbm.at[cid, pl.ds(off, ZR)])


def _node_body(x_ref, p0_ref, p1_ref, w1x_ref, w1a_ref, b1_ref, w2_ref,
               b2_ref, g_ref, bb_ref, o_ref):
    x = x_ref[...]
    aggr = p0_ref[...] + p1_ref[...]
    h1 = jnp.maximum(
        jnp.dot(x, w1x_ref[...], preferred_element_type=jnp.float32)
        + jnp.dot(aggr, w1a_ref[...], preferred_element_type=jnp.float32)
        + b1_ref[...], 0.0)
    h2 = jnp.dot(h1, w2_ref[...], preferred_element_type=jnp.float32) + b2_ref[...]
    mu = jnp.mean(h2, axis=1, keepdims=True)
    var = jnp.mean((h2 - mu) ** 2, axis=1, keepdims=True)
    o_ref[...] = (h2 - mu) / jnp.sqrt(var + EPS) * g_ref[...] + bb_ref[...] + x


def _node_mlp(x, p0, p1, w1x, w1a, b1, w2, b2, g, bb):
    full = lambda shape: pl.BlockSpec(shape, lambda i: (0,) * len(shape))
    row = pl.BlockSpec((_BN, D), lambda i: (i, 0))
    return pl.pallas_call(
        _node_body,
        grid=(N // _BN,),
        in_specs=[row, row, row,
                  full((D, D)), full((D, D)), full((1, D)), full((D, D)),
                  full((1, D)), full((1, D)), full((1, D))],
        out_specs=row,
        out_shape=jax.ShapeDtypeStruct((N, D), jnp.float32),
    )(x, p0, p1, w1x, w1a, b1, w2, b2, g, bb)


def kernel(x, g, pos, We1, be1, We2, be2, lne_w, lne_b,
           Wn1, bn1, Wn2, bn2, lnn_w, lnn_b):
    r = lambda v: v.reshape(1, D)
    wd = We1[0:P]
    wnrm = We1[P:P + 1]
    wxi = We1[P + 1:P + 1 + D]
    wxj = We1[P + 1 + D:]
    gi = g[0]
    gj = g[1]

    a_tab, b_tab = _prep(x, pos, wxi, wxj, wd, r(be1))
    pack = lambda t: jax.lax.bitcast_convert_type(
        t.reshape(N, D // 2, 2), jnp.int32)
    hab, n2 = _sc_gather(pack(a_tab), pack(b_tab), pos.reshape(-1), gi, gj)
    perm = jnp.arange(D).reshape(D // 2, 2).T.reshape(D)
    e_emb = _edge_tail(hab, n2.reshape(E, 1), wnrm[:, perm], We2[perm],
                       r(be2), r(lne_w), r(lne_b))
    parts = _sc_scatter(e_emb, gj)
    return _node_mlp(x, parts[0, :N], parts[1, :N], Wn1[:D], Wn1[D:], r(bn1),
                     Wn2, r(bn2), r(lnn_w), r(lnn_b))


# speedup vs baseline: 2.9238x; 1.0035x over previous
import functools

import jax
import jax.numpy as jnp
from jax import lax
from jax.experimental import pallas as pl
from jax.experimental.pallas import tpu as pltpu
from jax.experimental.pallas import tpu_sc as plsc

N = 10000
E = 320000
D = 128
P = 4
EPS = 1e-5

NC, NS, L = 2, 16, 16
NW = NC * NS
EPW = E // NW
C = 80
NCHUNK = EPW // C
NPAD = 10240
RPT = NPAD // NS
ZR = 128

_sc_mesh = plsc.VectorSubcoreMesh(
    core_axis_name="c", subcore_axis_name="s", num_cores=NC, num_subcores=NS)


def _prep_body(x_ref, pos_ref, wxi_ref, wxj_ref, wd_ref, b1_ref, a_ref, b_ref):
    x = x_ref[...]
    pd = jnp.dot(pos_ref[...], wd_ref[...], preferred_element_type=jnp.float32)
    a_ref[...] = (jnp.dot(x, wxi_ref[...], preferred_element_type=jnp.float32)
                  + pd + b1_ref[...]).astype(jnp.bfloat16)
    b_ref[...] = (jnp.dot(x, wxj_ref[...], preferred_element_type=jnp.float32)
                  - pd).astype(jnp.bfloat16)


_BN = 1000


def _prep(x, pos, wxi, wxj, wd, b1):
    full = lambda shape: pl.BlockSpec(shape, lambda i: (0,) * len(shape))
    return pl.pallas_call(
        _prep_body,
        grid=(N // _BN,),
        in_specs=[
            pl.BlockSpec((_BN, D), lambda i: (i, 0)),
            pl.BlockSpec((_BN, P), lambda i: (i, 0)),
            full((D, D)), full((D, D)), full((P, D)), full((1, D)),
        ],
        out_specs=[pl.BlockSpec((_BN, D), lambda i: (i, 0))] * 2,
        out_shape=[jax.ShapeDtypeStruct((N, D), jnp.bfloat16)] * 2,
    )(x, pos, wxi, wxj, wd, b1)


@functools.partial(
    pl.kernel,
    out_type=[jax.ShapeDtypeStruct((E, D), jnp.int32),
              jax.ShapeDtypeStruct((E,), jnp.float32)],
    mesh=_sc_mesh,
    scratch_types=[
        pltpu.VMEM((N * P,), jnp.float32),
        pltpu.VMEM((EPW,), jnp.int32),
        pltpu.VMEM((EPW,), jnp.int32),
        [pltpu.VMEM((C, D // 2), jnp.int32)] * 2,
        [pltpu.VMEM((C, D // 2), jnp.int32)] * 2,
        [pltpu.VMEM((C,), jnp.float32)] * 2,
        [pltpu.SemaphoreType.DMA] * 2,
        [pltpu.SemaphoreType.DMA] * 2,
        [pltpu.SemaphoreType.DMA] * 2,
    ],
    compiler_params=pltpu.CompilerParams(needs_layout_passes=False,
                                         use_tc_tiling_on_sc=False),
)
def _sc_gather(a_hbm, b_hbm, posf_hbm, gi_hbm, gj_hbm, hab_hbm, n2_hbm,
               posf_v, iv_all, jv_all, ra, rb, n2v, sem_a, sem_b, sem_w):
    wid = lax.axis_index("s") * NC + lax.axis_index("c")
    pltpu.sync_copy(posf_hbm, posf_v)
    pltpu.sync_copy(gi_hbm.at[pl.ds(wid * EPW, EPW)], iv_all)
    pltpu.sync_copy(gj_hbm.at[pl.ds(wid * EPW, EPW)], jv_all)

    def _issue(b, k):
        pltpu.async_copy(a_hbm.at[iv_all.at[pl.ds(k * C, C)]], ra[b], sem_a[b])
        pltpu.async_copy(b_hbm.at[jv_all.at[pl.ds(k * C, C)]], rb[b], sem_b[b])

    def _wait_gather(b):
        pltpu.make_async_copy(a_hbm.at[iv_all.at[pl.ds(0, C)]], ra[b], sem_a[b]).wait()
        pltpu.make_async_copy(b_hbm.at[jv_all.at[pl.ds(0, C)]], rb[b], sem_b[b]).wait()

    def _wait_write(b):
        pltpu.make_async_copy(
            ra[b], hab_hbm.at[pl.ds(0, C), pl.ds(0, D // 2)], sem_w[b]).wait()
        pltpu.make_async_copy(
            rb[b], hab_hbm.at[pl.ds(0, C), pl.ds(D // 2, D // 2)], sem_w[b]).wait()
        pltpu.make_async_copy(n2v[b], n2_hbm.at[pl.ds(0, C)], sem_w[b]).wait()

    def _compute(b, k):
        off = k * C
        for g_ in range(C // L):
            ivv = iv_all[pl.ds(off + g_ * L, L)] * P
            jvv = jv_all[pl.ds(off + g_ * L, L)] * P
            acc = jnp.zeros((L,), jnp.float32)
            for comp in range(P):
                pi = plsc.load_gather(posf_v, [ivv + comp])
                pj = plsc.load_gather(posf_v, [jvv + comp])
                dd = pi - pj
                acc = acc + dd * dd
            n2v[b][pl.ds(g_ * L, L)] = acc

        base = wid * EPW + off
        pltpu.async_copy(
            ra[b], hab_hbm.at[pl.ds(base, C), pl.ds(0, D // 2)], sem_w[b])
        pltpu.async_copy(
            rb[b], hab_hbm.at[pl.ds(base, C), pl.ds(D // 2, D // 2)], sem_w[b])
        pltpu.async_copy(n2v[b], n2_hbm.at[pl.ds(base, C)], sem_w[b])

    for b in range(2):
        _issue(b, b)

    @pl.loop(0, (NCHUNK - 1) // 2)
    def _pair(p):
        for b in range(2):
            k = 2 * p + b
            _wait_gather(b)
            _compute(b, k)
            _wait_write(b)
            if b == 0:
                _issue(0, k + 2)
            else:
                @pl.when(p < (NCHUNK - 1) // 2 - 1)
                def _():
                    _issue(1, k + 2)

    _wait_gather(0)
    _compute(0, NCHUNK - 1)
    _wait_write(0)


def _edge_body(hab_ref, n2_ref, wnrm_ref, we2_ref, b2_ref, g_ref,
               bb_ref, o_ref):
    w = hab_ref[...]
    a32 = w[:, :D // 2]
    b32 = w[:, D // 2:]
    asf = lambda v: jax.lax.bitcast_convert_type(v, jnp.float32)
    lo = lambda v: asf(jax.lax.shift_left(v, 16))
    hi = lambda v: asf(jax.lax.bitwise_and(v, jnp.int32(-65536)))
    h_pre = jnp.concatenate(
        [lo(a32) + lo(b32), hi(a32) + hi(b32)], axis=1)
    nrm = jnp.sqrt(n2_ref[...])
    h1 = jnp.maximum(h_pre + nrm * wnrm_ref[...], 0.0)
    h2 = jnp.dot(h1, we2_ref[...], preferred_element_type=jnp.float32) + b2_ref[...]
    mu = jnp.mean(h2, axis=1, keepdims=True)
    var = jnp.mean((h2 - mu) ** 2, axis=1, keepdims=True)
    o_ref[...] = (h2 - mu) / jnp.sqrt(var + EPS) * g_ref[...] + bb_ref[...]


_BE = 8000


def _edge_tail(hab, n2, wnrm, we2, b2, g, bb):
    full = lambda shape: pl.BlockSpec(shape, lambda i: (0,) * len(shape))
    return pl.pallas_call(
        _edge_body,
        grid=(E // _BE,),
        in_specs=[
            pl.BlockSpec((_BE, D), lambda i: (i, 0)),
            pl.BlockSpec((_BE, 1), lambda i: (i, 0)),
            full((1, D)), full((D, D)), full((1, D)), full((1, D)), full((1, D)),
        ],
        out_specs=pl.BlockSpec((_BE, D), lambda i: (i, 0)),
        out_shape=jax.ShapeDtypeStruct((E, D), jnp.float32),
    )(hab, n2, wnrm, we2, b2, g, bb)


@functools.partial(
    pl.kernel,
    out_type=jax.ShapeDtypeStruct((NC, NPAD, D), jnp.float32),
    mesh=_sc_mesh,
    scratch_types=[
        pltpu.VMEM_SHARED((NPAD, D), jnp.float32),
        [pltpu.VMEM((C,), jnp.int32)] * 2,
        [pltpu.VMEM((C, D), jnp.float32)] * 2,
        pltpu.VMEM((ZR, D), jnp.float32),
        [pltpu.SemaphoreType.DMA] * 2,
    ],
    compiler_params=pltpu.CompilerParams(needs_layout_passes=False),
)
def _sc_scatter(e_hbm, gj_hbm, out_hbm, aggr_s, jv, ebuf, zbuf, sem_e):
    cid = lax.axis_index("c")
    sid = lax.axis_index("s")
    wid = sid * NC + cid

    @pl.loop(0, ZR)
    def _z(r):
        for cc in range(D // L):
            zbuf[r, pl.ds(cc * L, L)] = jnp.zeros((L,), jnp.float32)

    for q in range(RPT // ZR):
        pltpu.sync_copy(zbuf, aggr_s.at[pl.ds(sid * RPT + q * ZR, ZR)])
    plsc.subcore_barrier()

    def _issue(b, k):
        base = wid * EPW + k * C
        pltpu.sync_copy(gj_hbm.at[pl.ds(base, C)], jv[b])
        pltpu.async_copy(e_hbm.at[pl.ds(base, C)], ebuf[b], sem_e[b])

    def _consume(b):
        pltpu.make_async_copy(e_hbm.at[pl.ds(0, C)], ebuf[b], sem_e[b]).wait()
        pltpu.sync_copy(ebuf[b], aggr_s.at[jv[b]], add=True)

    for b in range(2):
        _issue(b, b)

    @pl.loop(0, (NCHUNK - 1) // 2)
    def _pair(p):
        for b in range(2):
            k = 2 * p + b
            _consume(b)
            if b == 0:
                _issue(0, k + 2)
            else:
                @pl.when(p < (NCHUNK - 1) // 2 - 1)
                def _():
                    _issue(1, k + 2)

    _consume(0)
    plsc.subcore_barrier()
    for q in range(RPT // ZR):
        off = sid * RPT + q * ZR
        pltpu.sync_copy(aggr_s.at[pl.ds(off, ZR)], zbuf)
        pltpu.sync_copy(zbuf, out_hbm.at[cid, pl.ds(off, ZR)])


def _node_body(x_ref, p0_ref, p1_ref, w1x_ref, w1a_ref, b1_ref, w2_ref,
               b2_ref, g_ref, bb_ref, o_ref):
    x = x_ref[...]
    aggr = p0_ref[...] + p1_ref[...]
    h1 = jnp.maximum(
        jnp.dot(x, w1x_ref[...], preferred_element_type=jnp.float32)
        + jnp.dot(aggr, w1a_ref[...], preferred_element_type=jnp.float32)
        + b1_ref[...], 0.0)
    h2 = jnp.dot(h1, w2_ref[...], preferred_element_type=jnp.float32) + b2_ref[...]
    mu = jnp.mean(h2, axis=1, keepdims=True)
    var = jnp.mean((h2 - mu) ** 2, axis=1, keepdims=True)
    o_ref[...] = (h2 - mu) / jnp.sqrt(var + EPS) * g_ref[...] + bb_ref[...] + x


def _node_mlp(x, p0, p1, w1x, w1a, b1, w2, b2, g, bb):
    full = lambda shape: pl.BlockSpec(shape, lambda i: (0,) * len(shape))
    row = pl.BlockSpec((_BN, D), lambda i: (i, 0))
    return pl.pallas_call(
        _node_body,
        grid=(N // _BN,),
        in_specs=[row, row, row,
                  full((D, D)), full((D, D)), full((1, D)), full((D, D)),
                  full((1, D)), full((1, D)), full((1, D))],
        out_specs=row,
        out_shape=jax.ShapeDtypeStruct((N, D), jnp.float32),
    )(x, p0, p1, w1x, w1a, b1, w2, b2, g, bb)


def kernel(x, g, pos, We1, be1, We2, be2, lne_w, lne_b,
           Wn1, bn1, Wn2, bn2, lnn_w, lnn_b):
    r = lambda v: v.reshape(1, D)
    wd = We1[0:P]
    wnrm = We1[P:P + 1]
    wxi = We1[P + 1:P + 1 + D]
    wxj = We1[P + 1 + D:]
    gi = g[0]
    gj = g[1]

    a_tab, b_tab = _prep(x, pos, wxi, wxj, wd, r(be1))
    pack = lambda t: jax.lax.bitcast_convert_type(
        t.reshape(N, D // 2, 2), jnp.int32)
    hab, n2 = _sc_gather(pack(a_tab), pack(b_tab), pos.reshape(-1), gi, gj)
    perm = jnp.arange(D).reshape(D // 2, 2).T.reshape(D)
    e_emb = _edge_tail(hab, n2.reshape(E, 1), wnrm[:, perm], We2[perm],
                       r(be2), r(lne_w), r(lne_b))
    parts = _sc_scatter(e_emb, gj)
    return _node_mlp(x, parts[0, :N], parts[1, :N], Wn1[:D], Wn1[D:], r(bn1),
                     Wn2, r(bn2), r(lnn_w), r(lnn_b))
